# baseline (device time: 17500 ns/iter reference)
import jax
import jax.numpy as jnp
from jax import lax
from jax.experimental import pallas as pl
from jax.experimental.pallas import tpu as pltpu

N_DEV = 16
N_PLANE = 4
N_Z = 4


def kernel(x, router_W, route_idx, expert_W):
    m, d_model = x.shape
    n_local, _, h = expert_W.shape
    rows_per = m // N_DEV
    slab = m // N_Z

    def body(x_ref, rw_ref, idx_ref, ew_ref, out_ref,
             xs0_ref, xs1_ref, pbf_ref, rs_recv,
             send_sems, recv_sems):
        my = lax.axis_index("i")
        g = my // N_PLANE
        w = my % N_PLANE

        bsem = pltpu.get_barrier_semaphore()
        for k in range(1, N_DEV):
            pl.semaphore_signal(
                bsem, inc=1,
                device_id=((my + k) % N_DEV,),
                device_id_type=pl.DeviceIdType.MESH,
            )
        pl.semaphore_wait(bsem, N_DEV - 1)

        xv = x_ref[:, :]
        scores = jnp.dot(xv, rw_ref[:, :], preferred_element_type=jnp.float32)
        s_max = jnp.max(scores, axis=1, keepdims=True)
        e = jnp.exp(scores - s_max)
        probs = e / jnp.sum(e, axis=1, keepdims=True)
        e_ids = lax.broadcasted_iota(jnp.int32, scores.shape, 1)
        idx0 = idx_ref[:, 0:1]
        idx1 = idx_ref[:, 1:2]
        p0 = jnp.sum(jnp.where(e_ids == idx0, probs, 0.0), axis=1,
                     keepdims=True)
        p1 = jnp.sum(jnp.where(e_ids == idx1, probs, 0.0), axis=1,
                     keepdims=True)
        denom = p0 + p1

        for j, xs_ref in ((0, xs0_ref), (1, xs1_ref)):
            gid = 2 * my + j
            wj = (jnp.where(idx0 == gid, p0, 0.0)
                  + jnp.where(idx1 == gid, p1, 0.0)) / denom
            xs_ref[:, :] = (xv * wj).astype(jnp.bfloat16)
        wm0 = ew_ref[0].astype(jnp.bfloat16)
        wm1 = ew_ref[1].astype(jnp.bfloat16)

        sends = []
        for zk in range(N_Z):
            z = (g + 1 + zk) % N_Z
            s0 = z * slab
            a0 = xs0_ref[pl.ds(s0, slab), :]
            a1 = xs1_ref[pl.ds(s0, slab), :]
            sl = (jnp.dot(a0, wm0, preferred_element_type=jnp.float32)
                  + jnp.dot(a1, wm1, preferred_element_type=jnp.float32))
            pbf_ref[pl.ds(s0, slab), :] = sl.astype(jnp.bfloat16)
            n_dst = N_PLANE if zk < N_Z - 1 else N_PLANE - 1
            for k in range(n_dst):
                w2 = (w + k + (N_Z - 1 - zk == 0)) % N_PLANE
                q = z * N_PLANE + w2
                rdma = pltpu.make_async_remote_copy(
                    src_ref=pbf_ref.at[pl.ds(q * rows_per, rows_per), :],
                    dst_ref=rs_recv.at[my],
                    send_sem=send_sems.at[zk, k],
                    recv_sem=recv_sems.at[my],
                    device_id=(q,),
                    device_id_type=pl.DeviceIdType.MESH,
                )
                rdma.start()
                sends.append(rdma)

        acc = pbf_ref[pl.ds(my * rows_per, rows_per), :].astype(jnp.float32)
        for k in range(1, N_DEV):
            p = (my + k) % N_DEV
            recv = pltpu.make_async_remote_copy(
                src_ref=pbf_ref.at[pl.ds(0, rows_per), :],
                dst_ref=rs_recv.at[p],
                send_sem=send_sems.at[0, 0],
                recv_sem=recv_sems.at[p],
                device_id=(my,),
                device_id_type=pl.DeviceIdType.MESH,
            )
            recv.wait_recv()
            acc = acc + rs_recv[p].astype(jnp.float32)
        out_ref[:, :] = acc

        for rdma in sends:
            rdma.wait_send()

    return pl.pallas_call(
        body,
        out_shape=jax.ShapeDtypeStruct((rows_per, h), jnp.float32),
        in_specs=[
            pl.BlockSpec(memory_space=pltpu.VMEM),
            pl.BlockSpec(memory_space=pltpu.VMEM),
            pl.BlockSpec(memory_space=pltpu.VMEM),
            pl.BlockSpec(memory_space=pltpu.VMEM),
        ],
        out_specs=pl.BlockSpec(memory_space=pltpu.VMEM),
        scratch_shapes=[
            pltpu.VMEM((m, d_model), jnp.bfloat16),
            pltpu.VMEM((m, d_model), jnp.bfloat16),
            pltpu.VMEM((m, h), jnp.bfloat16),
            pltpu.VMEM((N_DEV, rows_per, h), jnp.bfloat16),
            pltpu.SemaphoreType.DMA((N_Z, N_PLANE)),
            pltpu.SemaphoreType.DMA((N_DEV,)),
        ],
        compiler_params=pltpu.CompilerParams(collective_id=0),
    )(x, router_W, route_idx, expert_W)


# device time: 15737 ns/iter; 1.1120x vs baseline; 1.1120x over previous
import jax
import jax.numpy as jnp
from jax import lax
from jax.experimental import pallas as pl
from jax.experimental.pallas import tpu as pltpu

N_DEV = 16
N_PLANE = 4
N_Z = 4


def kernel(x, router_W, route_idx, expert_W):
    m, d_model = x.shape
    n_local, _, h = expert_W.shape
    rows_per = m // N_DEV
    slab = m // N_Z

    def body(x_ref, rw_ref, idx_ref, ew_ref, out_ref,
             w0_ref, w1_ref, pbf_ref, colacc_ref, rs1_recv, rs2_recv,
             sa_send_sems, sa_recv_sems, sb_send_sems, sb_recv_sems):
        my = lax.axis_index("i")
        g = my // N_PLANE
        w = my % N_PLANE

        bsem = pltpu.get_barrier_semaphore()
        for k in range(1, N_PLANE):
            pl.semaphore_signal(
                bsem, inc=1,
                device_id=(g * N_PLANE + (w + k) % N_PLANE,),
                device_id_type=pl.DeviceIdType.MESH,
            )
        for k in range(1, N_Z):
            pl.semaphore_signal(
                bsem, inc=1,
                device_id=(((g + k) % N_Z) * N_PLANE + w,),
                device_id_type=pl.DeviceIdType.MESH,
            )

        xv = x_ref[:, :]
        scores = jnp.dot(xv, rw_ref[:, :], preferred_element_type=jnp.float32)
        e_ids = lax.broadcasted_iota(jnp.int32, scores.shape, 1)
        idx0 = idx_ref[:, 0:1]
        idx1 = idx_ref[:, 1:2]
        s0 = jnp.sum(jnp.where(e_ids == idx0, scores, 0.0), axis=1,
                     keepdims=True)
        s1 = jnp.sum(jnp.where(e_ids == idx1, scores, 0.0), axis=1,
                     keepdims=True)
        gate0 = 1.0 / (1.0 + jnp.exp(s1 - s0))
        gid0 = 2 * my
        gid1 = 2 * my + 1
        w0_ref[:, :] = (jnp.where(idx0 == gid0, gate0, 0.0)
                        + jnp.where(idx1 == gid0, 1.0 - gate0, 0.0))
        w1_ref[:, :] = (jnp.where(idx0 == gid1, gate0, 0.0)
                        + jnp.where(idx1 == gid1, 1.0 - gate0, 0.0))
        wm0 = ew_ref[0].astype(jnp.bfloat16)
        wm1 = ew_ref[1].astype(jnp.bfloat16)

        sends = []
        for zk in range(N_Z):
            z = (g + 1 + zk) % N_Z
            s_row = z * slab
            xsl = x_ref[pl.ds(s_row, slab), :]
            a0 = (xsl * w0_ref[pl.ds(s_row, slab), :]).astype(jnp.bfloat16)
            a1 = (xsl * w1_ref[pl.ds(s_row, slab), :]).astype(jnp.bfloat16)
            sl = (jnp.dot(a0, wm0, preferred_element_type=jnp.float32)
                  + jnp.dot(a1, wm1, preferred_element_type=jnp.float32))
            pbf_ref[pl.ds(s_row, slab), :] = sl.astype(jnp.bfloat16)
            if zk == 0:
                pl.semaphore_wait(bsem, N_PLANE - 1 + N_Z - 1)
            for k in range(1, N_PLANE):
                w2 = (w + k) % N_PLANE
                rdma = pltpu.make_async_remote_copy(
                    src_ref=pbf_ref.at[
                        pl.ds((z * N_PLANE + w2) * rows_per, rows_per), :],
                    dst_ref=rs1_recv.at[w, z],
                    send_sem=sa_send_sems.at[k, zk],
                    recv_sem=sa_recv_sems.at[w, z],
                    device_id=(g * N_PLANE + w2,),
                    device_id_type=pl.DeviceIdType.MESH,
                )
                rdma.start()
                sends.append(rdma)

        own_blk = None
        for zk in range(N_Z):
            g2 = (g + 1 + zk) % N_Z
            blk = pbf_ref[pl.ds((g2 * N_PLANE + w) * rows_per,
                                rows_per), :].astype(jnp.float32)
            for k in range(1, N_PLANE):
                pw = (w + k) % N_PLANE
                recv = pltpu.make_async_remote_copy(
                    src_ref=pbf_ref.at[pl.ds(0, rows_per), :],
                    dst_ref=rs1_recv.at[pw, g2],
                    send_sem=sa_send_sems.at[k, zk],
                    recv_sem=sa_recv_sems.at[pw, g2],
                    device_id=(my,),
                    device_id_type=pl.DeviceIdType.MESH,
                )
                recv.wait_recv()
                blk = blk + rs1_recv[pw, g2].astype(jnp.float32)
            if zk < N_Z - 1:
                colacc_ref[zk] = blk.astype(jnp.bfloat16)
                rdma = pltpu.make_async_remote_copy(
                    src_ref=colacc_ref.at[zk],
                    dst_ref=rs2_recv.at[g],
                    send_sem=sb_send_sems.at[zk],
                    recv_sem=sb_recv_sems.at[g],
                    device_id=(g2 * N_PLANE + w,),
                    device_id_type=pl.DeviceIdType.MESH,
                )
                rdma.start()
                sends.append(rdma)
            else:
                own_blk = blk

        acc = own_blk
        for k in range(1, N_Z):
            pg = (g + k) % N_Z
            recv = pltpu.make_async_remote_copy(
                src_ref=colacc_ref.at[0],
                dst_ref=rs2_recv.at[pg],
                send_sem=sb_send_sems.at[0],
                recv_sem=sb_recv_sems.at[pg],
                device_id=(my,),
                device_id_type=pl.DeviceIdType.MESH,
            )
            recv.wait_recv()
            acc = acc + rs2_recv[pg].astype(jnp.float32)
        out_ref[:, :] = acc

        for rdma in sends:
            rdma.wait_send()

    return pl.pallas_call(
        body,
        out_shape=jax.ShapeDtypeStruct((rows_per, h), jnp.float32),
        in_specs=[
            pl.BlockSpec(memory_space=pltpu.VMEM),
            pl.BlockSpec(memory_space=pltpu.VMEM),
            pl.BlockSpec(memory_space=pltpu.VMEM),
            pl.BlockSpec(memory_space=pltpu.VMEM),
        ],
        out_specs=pl.BlockSpec(memory_space=pltpu.VMEM),
        scratch_shapes=[
            pltpu.VMEM((m, 1), jnp.float32),
            pltpu.VMEM((m, 1), jnp.float32),
            pltpu.VMEM((m, h), jnp.bfloat16),
            pltpu.VMEM((N_Z - 1, rows_per, h), jnp.bfloat16),
            pltpu.VMEM((N_PLANE, N_Z, rows_per, h), jnp.bfloat16),
            pltpu.VMEM((N_Z, rows_per, h), jnp.bfloat16),
            pltpu.SemaphoreType.DMA((N_PLANE, N_Z)),
            pltpu.SemaphoreType.DMA((N_PLANE, N_Z)),
            pltpu.SemaphoreType.DMA((N_Z,)),
            pltpu.SemaphoreType.DMA((N_Z,)),
        ],
        compiler_params=pltpu.CompilerParams(collective_id=0),
    )(x, router_W, route_idx, expert_W)


# device time: 14823 ns/iter; 1.1806x vs baseline; 1.0617x over previous
import jax
import jax.numpy as jnp
from jax import lax
from jax.experimental import pallas as pl
from jax.experimental.pallas import tpu as pltpu

N_DEV = 16
N_PLANE = 4
N_Z = 4


def kernel(x, router_W, route_idx, expert_W):
    m, d_model = x.shape
    n_local, _, h = expert_W.shape
    rows_per = m // N_DEV
    slab = m // N_Z

    def body(x_ref, rw_ref, idx_ref, ew_ref, out_ref,
             w0_ref, w1_ref, pbf_ref, colacc_ref, rs1_recv, rs2_recv,
             sa_send_sems, sa_recv_sems, sb_send_sems, sb_recv_sems):
        my = lax.axis_index("i")
        g = my // N_PLANE
        w = my % N_PLANE

        bsem = pltpu.get_barrier_semaphore()
        for k in range(1, N_PLANE):
            pl.semaphore_signal(
                bsem, inc=1,
                device_id=(g * N_PLANE + (w + k) % N_PLANE,),
                device_id_type=pl.DeviceIdType.MESH,
            )
        for k in range(1, N_Z):
            pl.semaphore_signal(
                bsem, inc=1,
                device_id=(((g + k) % N_Z) * N_PLANE + w,),
                device_id_type=pl.DeviceIdType.MESH,
            )

        xv = x_ref[:, :]
        scores = jnp.dot(xv, rw_ref[:, :], preferred_element_type=jnp.float32)
        e_ids = lax.broadcasted_iota(jnp.int32, scores.shape, 1)
        idx0 = idx_ref[:, 0:1]
        idx1 = idx_ref[:, 1:2]
        s0 = jnp.sum(jnp.where(e_ids == idx0, scores, 0.0), axis=1,
                     keepdims=True)
        s1 = jnp.sum(jnp.where(e_ids == idx1, scores, 0.0), axis=1,
                     keepdims=True)
        gate0 = 1.0 / (1.0 + jnp.exp(s1 - s0))
        gid0 = 2 * my
        gid1 = 2 * my + 1
        w0_ref[:, :] = (jnp.where(idx0 == gid0, gate0, 0.0)
                        + jnp.where(idx1 == gid0, 1.0 - gate0, 0.0))
        w1_ref[:, :] = (jnp.where(idx0 == gid1, gate0, 0.0)
                        + jnp.where(idx1 == gid1, 1.0 - gate0, 0.0))
        wm0 = ew_ref[0][:, :]
        wm1 = ew_ref[1][:, :]

        sends = []
        for zk in range(N_Z):
            z = (g + 1 + zk) % N_Z
            s_row = z * slab
            xsl = x_ref[pl.ds(s_row, slab), :]
            a0 = (xsl * w0_ref[pl.ds(s_row, slab), :]).astype(jnp.bfloat16)
            a1 = (xsl * w1_ref[pl.ds(s_row, slab), :]).astype(jnp.bfloat16)
            sl = (jnp.dot(a0, wm0, preferred_element_type=jnp.float32)
                  + jnp.dot(a1, wm1, preferred_element_type=jnp.float32))
            pbf_ref[pl.ds(s_row, slab), :] = sl.astype(jnp.bfloat16)
            if zk == 0:
                pl.semaphore_wait(bsem, N_PLANE - 1 + N_Z - 1)
            for k in range(1, N_PLANE):
                w2 = (w + k) % N_PLANE
                rdma = pltpu.make_async_remote_copy(
                    src_ref=pbf_ref.at[
                        pl.ds((z * N_PLANE + w2) * rows_per, rows_per), :],
                    dst_ref=rs1_recv.at[w, z],
                    send_sem=sa_send_sems.at[k, zk],
                    recv_sem=sa_recv_sems.at[w, z],
                    device_id=(g * N_PLANE + w2,),
                    device_id_type=pl.DeviceIdType.MESH,
                )
                rdma.start()
                sends.append(rdma)

        own_blk = None
        for zk in range(N_Z):
            g2 = (g + 1 + zk) % N_Z
            blk = pbf_ref[pl.ds((g2 * N_PLANE + w) * rows_per,
                                rows_per), :].astype(jnp.float32)
            for k in range(1, N_PLANE):
                pw = (w + k) % N_PLANE
                recv = pltpu.make_async_remote_copy(
                    src_ref=pbf_ref.at[pl.ds(0, rows_per), :],
                    dst_ref=rs1_recv.at[pw, g2],
                    send_sem=sa_send_sems.at[k, zk],
                    recv_sem=sa_recv_sems.at[pw, g2],
                    device_id=(my,),
                    device_id_type=pl.DeviceIdType.MESH,
                )
                recv.wait_recv()
                blk = blk + rs1_recv[pw, g2].astype(jnp.float32)
            if zk < N_Z - 1:
                colacc_ref[zk] = blk.astype(jnp.bfloat16)
                rdma = pltpu.make_async_remote_copy(
                    src_ref=colacc_ref.at[zk],
                    dst_ref=rs2_recv.at[g],
                    send_sem=sb_send_sems.at[zk],
                    recv_sem=sb_recv_sems.at[g],
                    device_id=(g2 * N_PLANE + w,),
                    device_id_type=pl.DeviceIdType.MESH,
                )
                rdma.start()
                sends.append(rdma)
            else:
                own_blk = blk

        acc = own_blk
        for k in range(1, N_Z):
            pg = (g + k) % N_Z
            recv = pltpu.make_async_remote_copy(
                src_ref=colacc_ref.at[0],
                dst_ref=rs2_recv.at[pg],
                send_sem=sb_send_sems.at[0],
                recv_sem=sb_recv_sems.at[pg],
                device_id=(my,),
                device_id_type=pl.DeviceIdType.MESH,
            )
            recv.wait_recv()
            acc = acc + rs2_recv[pg].astype(jnp.float32)
        out_ref[:, :] = acc

        for rdma in sends:
            rdma.wait_send()

    return pl.pallas_call(
        body,
        out_shape=jax.ShapeDtypeStruct((rows_per, h), jnp.float32),
        in_specs=[
            pl.BlockSpec(memory_space=pltpu.VMEM),
            pl.BlockSpec(memory_space=pltpu.VMEM),
            pl.BlockSpec(memory_space=pltpu.VMEM),
            pl.BlockSpec(memory_space=pltpu.VMEM),
        ],
        out_specs=pl.BlockSpec(memory_space=pltpu.VMEM),
        scratch_shapes=[
            pltpu.VMEM((m, 1), jnp.float32),
            pltpu.VMEM((m, 1), jnp.float32),
            pltpu.VMEM((m, h), jnp.bfloat16),
            pltpu.VMEM((N_Z - 1, rows_per, h), jnp.bfloat16),
            pltpu.VMEM((N_PLANE, N_Z, rows_per, h), jnp.bfloat16),
            pltpu.VMEM((N_Z, rows_per, h), jnp.bfloat16),
            pltpu.SemaphoreType.DMA((N_PLANE, N_Z)),
            pltpu.SemaphoreType.DMA((N_PLANE, N_Z)),
            pltpu.SemaphoreType.DMA((N_Z,)),
            pltpu.SemaphoreType.DMA((N_Z,)),
        ],
        compiler_params=pltpu.CompilerParams(collective_id=0),
    )(x, router_W, route_idx, expert_W.astype(jnp.bfloat16))
